# baseline (speedup 1.0000x reference)
import functools

import jax
import jax.numpy as jnp
from jax import lax
from jax.experimental import pallas as pl
from jax.experimental.pallas import tpu as pltpu
from jax.experimental.pallas import tpu_sc as plsc

N = 10000
E = 320000
F_IN = 128
HID = 128
CLS = 64

NPAD = 10240
K = 128
NTILES = 32
STEPS = 80
EPAD = NTILES * STEPS * K
ROWS = EPAD // K
S0, S1 = 80, 80
SMAX = max(S0, S1)
RPT = NPAD // 16

_MESH = dict(core_axis_name="c", subcore_axis_name="s")

ROW_BLK = 256
GRID = NPAD // ROW_BLK


def _deg_kernel(dstp, ones16, zeros16):
    @functools.partial(
        pl.kernel,
        out_type=jax.ShapeDtypeStruct((2, NPAD, 16), jnp.float32),
        mesh=plsc.VectorSubcoreMesh(**_MESH),
        compiler_params=pltpu.CompilerParams(use_tc_tiling_on_sc=False),
        scratch_types=[
            pltpu.VMEM((STEPS, K), jnp.int32),
            pltpu.VMEM((K, 16), jnp.float32),
            pltpu.VMEM_SHARED((NPAD, 16), jnp.float32),
        ],
    )
    def body(dst_hbm, ones_hbm, zeros_hbm, out_hbm, dstx, ones_v, acc):
        c = lax.axis_index("c")
        s = lax.axis_index("s")
        wid = s * 2 + c
        r0 = s * RPT
        pltpu.sync_copy(zeros_hbm.at[pl.ds(r0, RPT)], acc.at[pl.ds(r0, RPT)])
        pltpu.sync_copy(ones_hbm, ones_v)
        pltpu.sync_copy(dst_hbm.at[pl.ds(wid * STEPS, STEPS)], dstx)
        plsc.subcore_barrier()

        def step(j, carry):
            pltpu.sync_copy(ones_v, acc.at[dstx.at[j]], add=True)
            return carry

        lax.fori_loop(0, STEPS, step, 0)
        plsc.subcore_barrier()
        pltpu.sync_copy(acc.at[pl.ds(r0, RPT)], out_hbm.at[c, pl.ds(r0, RPT)])

    return body(dstp, ones16, zeros16)


def _make_spmm(width):
    @functools.partial(
        pl.kernel,
        out_type=jax.ShapeDtypeStruct((2, NPAD, width), jnp.float32),
        mesh=plsc.VectorSubcoreMesh(**_MESH),
        compiler_params=pltpu.CompilerParams(use_tc_tiling_on_sc=False),
        scratch_types=[
            pltpu.VMEM((SMAX, K), jnp.int32),
            pltpu.VMEM((SMAX, K), jnp.int32),
            pltpu.VMEM((K, width), jnp.float32),
            pltpu.VMEM_SHARED((NPAD, width), jnp.float32),
            pltpu.SemaphoreType.DMA,
        ],
    )
    def body(g_hbm, src_hbm, dst_hbm, zeros_hbm, out_hbm, srcx, dstx, buf, acc,
             sem):
        c = lax.axis_index("c")
        s = lax.axis_index("s")
        r0 = s * RPT
        nsteps = jnp.where(c == 0, S0, S1)
        base = jnp.where(c == 0, s * S0, 16 * S0 + s * S1)
        pltpu.sync_copy(zeros_hbm.at[pl.ds(r0, RPT)], acc.at[pl.ds(r0, RPT)])
        pltpu.sync_copy(src_hbm.at[pl.ds(base, SMAX)], srcx)
        pltpu.sync_copy(dst_hbm.at[pl.ds(base, SMAX)], dstx)
        plsc.subcore_barrier()

        def step(j, carry):
            pltpu.async_copy(g_hbm.at[srcx.at[j]], buf, sem).wait()
            pltpu.sync_copy(buf, acc.at[dstx.at[j]], add=True)
            return carry

        lax.fori_loop(0, nsteps, step, 0)
        plsc.subcore_barrier()
        pltpu.sync_copy(acc.at[pl.ds(r0, RPT)], out_hbm.at[c, pl.ds(r0, RPT)])

    return body


_spmm128 = _make_spmm(HID)
_spmm64 = _make_spmm(CLS)


def _deg_dinv(degp):
    deg = 1.0 + jnp.sum(degp[0], axis=1) + jnp.sum(degp[1], axis=1)
    dinv = lax.rsqrt(deg)
    return deg, dinv


def _tc1_body(degp_ref, x_ref, w1_ref, g1_ref):
    _, dinv = _deg_dinv(degp_ref[...])
    g1_ref[...] = jnp.dot(dinv[:, None] * x_ref[...], w1_ref[...],
                          preferred_element_type=jnp.float32)


def _tc2_body(degp_ref, part_ref, g1_ref, w2_ref, b1_ref, g2_ref):
    deg, dinv = _deg_dinv(degp_ref[...])
    p = part_ref[...]
    ssum = p[0] + p[1] + g1_ref[...]
    h1 = jnp.maximum((dinv / deg)[:, None] * ssum + b1_ref[...], 0.0)
    g2_ref[...] = dinv[:, None] * jnp.dot(h1, w2_ref[...],
                                          preferred_element_type=jnp.float32)


def _tc3_body(degp_ref, part_ref, g2_ref, b2_ref, out_ref):
    deg, dinv = _deg_dinv(degp_ref[...])
    p = part_ref[...]
    a = (dinv / deg)[:, None] * (p[0] + p[1] + g2_ref[...]) + b2_ref[...]
    m = jnp.max(a, axis=1, keepdims=True)
    ex = jnp.exp(a - m)
    out_ref[...] = (a - m) - jnp.log(jnp.sum(ex, axis=1, keepdims=True))


def _degp_spec():
    return pl.BlockSpec((2, ROW_BLK, 16), lambda i: (0, i, 0))


def _tc1(degp, xp, W1):
    return pl.pallas_call(
        _tc1_body,
        grid=(GRID,),
        in_specs=[
            _degp_spec(),
            pl.BlockSpec((ROW_BLK, F_IN), lambda i: (i, 0)),
            pl.BlockSpec((F_IN, HID), lambda i: (0, 0)),
        ],
        out_specs=pl.BlockSpec((ROW_BLK, HID), lambda i: (i, 0)),
        out_shape=jax.ShapeDtypeStruct((NPAD, HID), jnp.float32),
    )(degp, xp, W1)


def _tc2(degp, part1, g1, W2, b1):
    return pl.pallas_call(
        _tc2_body,
        grid=(GRID,),
        in_specs=[
            _degp_spec(),
            pl.BlockSpec((2, ROW_BLK, HID), lambda i: (0, i, 0)),
            pl.BlockSpec((ROW_BLK, HID), lambda i: (i, 0)),
            pl.BlockSpec((HID, CLS), lambda i: (0, 0)),
            pl.BlockSpec((1, HID), lambda i: (0, 0)),
        ],
        out_specs=pl.BlockSpec((ROW_BLK, CLS), lambda i: (i, 0)),
        out_shape=jax.ShapeDtypeStruct((NPAD, CLS), jnp.float32),
    )(degp, part1, g1, W2, b1)


def _tc3(degp, part2, g2, b2):
    blk = 400
    return pl.pallas_call(
        _tc3_body,
        grid=(N // blk,),
        in_specs=[
            pl.BlockSpec((2, blk, 16), lambda i: (0, i, 0)),
            pl.BlockSpec((2, blk, CLS), lambda i: (0, i, 0)),
            pl.BlockSpec((blk, CLS), lambda i: (i, 0)),
            pl.BlockSpec((1, CLS), lambda i: (0, 0)),
        ],
        out_specs=pl.BlockSpec((blk, CLS), lambda i: (i, 0)),
        out_shape=jax.ShapeDtypeStruct((N, CLS), jnp.float32),
    )(degp, part2, g2, b2)


def kernel(x, edge_index, W1, b1, W2, b2):
    pad = EPAD - E
    padv = jnp.full((pad,), N, dtype=jnp.int32)
    srcp = jnp.concatenate([edge_index[0], padv]).reshape(ROWS, K)
    dstp = jnp.concatenate([edge_index[1], padv]).reshape(ROWS, K)
    xp = jnp.zeros((NPAD, F_IN), jnp.float32).at[:N].set(x)

    ones16 = jnp.ones((K, 16), jnp.float32)
    zeros16 = jnp.zeros((NPAD, 16), jnp.float32)
    zeros128 = jnp.zeros((NPAD, HID), jnp.float32)
    zeros64 = jnp.zeros((NPAD, CLS), jnp.float32)

    degp = _deg_kernel(dstp, ones16, zeros16)
    g1 = _tc1(degp, xp, W1)
    part1 = _spmm128(g1, srcp, dstp, zeros128)
    g2 = _tc2(degp, part1, g1, W2, b1.reshape(1, HID))
    part2 = _spmm64(g2, srcp, dstp, zeros64)
    return _tc3(degp, part2, g2, b2.reshape(1, CLS))


# speedup vs baseline: 1.0006x; 1.0006x over previous
import functools

import jax
import jax.numpy as jnp
from jax import lax
from jax.experimental import pallas as pl
from jax.experimental.pallas import tpu as pltpu
from jax.experimental.pallas import tpu_sc as plsc

N = 10000
E = 320000
F_IN = 128
HID = 128
CLS = 64

NPAD = 10240
K = 128
NTILES = 32
STEPS = 80
EPAD = NTILES * STEPS * K
ROWS = EPAD // K
S0, S1 = 80, 80
SMAX = max(S0, S1)
RPT = NPAD // 16

_MESH = dict(core_axis_name="c", subcore_axis_name="s")

ROW_BLK = 256
GRID = NPAD // ROW_BLK


def _deg_kernel(dstp, ones16, zeros16):
    @functools.partial(
        pl.kernel,
        out_type=jax.ShapeDtypeStruct((2, NPAD, 16), jnp.float32),
        mesh=plsc.VectorSubcoreMesh(**_MESH),
        compiler_params=pltpu.CompilerParams(use_tc_tiling_on_sc=False),
        scratch_types=[
            pltpu.VMEM((STEPS, K), jnp.int32),
            pltpu.VMEM((K, 16), jnp.float32),
            pltpu.VMEM_SHARED((NPAD, 16), jnp.float32),
        ],
    )
    def body(dst_hbm, ones_hbm, zeros_hbm, out_hbm, dstx, ones_v, acc):
        c = lax.axis_index("c")
        s = lax.axis_index("s")
        wid = s * 2 + c
        r0 = s * RPT
        pltpu.sync_copy(zeros_hbm.at[pl.ds(r0, RPT)], acc.at[pl.ds(r0, RPT)])
        pltpu.sync_copy(ones_hbm, ones_v)
        pltpu.sync_copy(dst_hbm.at[pl.ds(wid * STEPS, STEPS)], dstx)
        plsc.subcore_barrier()

        def step(j, carry):
            pltpu.sync_copy(ones_v, acc.at[dstx.at[j]], add=True)
            return carry

        lax.fori_loop(0, STEPS, step, 0)
        plsc.subcore_barrier()
        pltpu.sync_copy(acc.at[pl.ds(r0, RPT)], out_hbm.at[c, pl.ds(r0, RPT)])

    return body(dstp, ones16, zeros16)


def _make_spmm(width):
    @functools.partial(
        pl.kernel,
        out_type=jax.ShapeDtypeStruct((2, NPAD, width), jnp.float32),
        mesh=plsc.VectorSubcoreMesh(**_MESH),
        compiler_params=pltpu.CompilerParams(use_tc_tiling_on_sc=False),
        scratch_types=[
            pltpu.VMEM((SMAX, K), jnp.int32),
            pltpu.VMEM((SMAX, K), jnp.int32),
            pltpu.VMEM((K, width), jnp.float32),
            pltpu.VMEM_SHARED((NPAD, width), jnp.float32),
            pltpu.SemaphoreType.DMA,
        ],
    )
    def body(g_hbm, src_hbm, dst_hbm, zeros_hbm, out_hbm, srcx, dstx, buf, acc,
             sem):
        c = lax.axis_index("c")
        s = lax.axis_index("s")
        r0 = s * RPT
        base = jnp.where(c == 0, s * S0, 16 * S0 + s * S1)
        pltpu.sync_copy(zeros_hbm.at[pl.ds(r0, RPT)], acc.at[pl.ds(r0, RPT)])
        pltpu.sync_copy(src_hbm.at[pl.ds(base, SMAX)], srcx)
        pltpu.sync_copy(dst_hbm.at[pl.ds(base, SMAX)], dstx)
        plsc.subcore_barrier()

        def step(j, carry):
            pltpu.async_copy(g_hbm.at[srcx.at[j]], buf, sem).wait()
            pltpu.sync_copy(buf, acc.at[dstx.at[j]], add=True)
            return carry

        lax.fori_loop(0, S0, step, 0)
        plsc.subcore_barrier()
        pltpu.sync_copy(acc.at[pl.ds(r0, RPT)], out_hbm.at[c, pl.ds(r0, RPT)])

    return body


_spmm128 = _make_spmm(HID)
_spmm64 = _make_spmm(CLS)


def _deg_dinv(degp):
    deg = 1.0 + jnp.sum(degp[0], axis=1) + jnp.sum(degp[1], axis=1)
    dinv = lax.rsqrt(deg)
    return deg, dinv


def _tc1_body(degp_ref, x_ref, w1_ref, g1_ref):
    _, dinv = _deg_dinv(degp_ref[...])
    g1_ref[...] = jnp.dot(dinv[:, None] * x_ref[...], w1_ref[...],
                          preferred_element_type=jnp.float32)


def _tc2_body(degp_ref, part_ref, g1_ref, w2_ref, b1_ref, g2_ref):
    deg, dinv = _deg_dinv(degp_ref[...])
    p = part_ref[...]
    ssum = p[0] + p[1] + g1_ref[...]
    h1 = jnp.maximum((dinv / deg)[:, None] * ssum + b1_ref[...], 0.0)
    g2_ref[...] = dinv[:, None] * jnp.dot(h1, w2_ref[...],
                                          preferred_element_type=jnp.float32)


def _tc3_body(degp_ref, part_ref, g2_ref, b2_ref, out_ref):
    deg, dinv = _deg_dinv(degp_ref[...])
    p = part_ref[...]
    a = (dinv / deg)[:, None] * (p[0] + p[1] + g2_ref[...]) + b2_ref[...]
    m = jnp.max(a, axis=1, keepdims=True)
    ex = jnp.exp(a - m)
    out_ref[...] = (a - m) - jnp.log(jnp.sum(ex, axis=1, keepdims=True))


def _degp_spec():
    return pl.BlockSpec((2, ROW_BLK, 16), lambda i: (0, i, 0))


def _tc1(degp, xp, W1):
    return pl.pallas_call(
        _tc1_body,
        grid=(GRID,),
        in_specs=[
            _degp_spec(),
            pl.BlockSpec((ROW_BLK, F_IN), lambda i: (i, 0)),
            pl.BlockSpec((F_IN, HID), lambda i: (0, 0)),
        ],
        out_specs=pl.BlockSpec((ROW_BLK, HID), lambda i: (i, 0)),
        out_shape=jax.ShapeDtypeStruct((NPAD, HID), jnp.float32),
    )(degp, xp, W1)


def _tc2(degp, part1, g1, W2, b1):
    return pl.pallas_call(
        _tc2_body,
        grid=(GRID,),
        in_specs=[
            _degp_spec(),
            pl.BlockSpec((2, ROW_BLK, HID), lambda i: (0, i, 0)),
            pl.BlockSpec((ROW_BLK, HID), lambda i: (i, 0)),
            pl.BlockSpec((HID, CLS), lambda i: (0, 0)),
            pl.BlockSpec((1, HID), lambda i: (0, 0)),
        ],
        out_specs=pl.BlockSpec((ROW_BLK, CLS), lambda i: (i, 0)),
        out_shape=jax.ShapeDtypeStruct((NPAD, CLS), jnp.float32),
    )(degp, part1, g1, W2, b1)


def _tc3(degp, part2, g2, b2):
    blk = 400
    return pl.pallas_call(
        _tc3_body,
        grid=(N // blk,),
        in_specs=[
            pl.BlockSpec((2, blk, 16), lambda i: (0, i, 0)),
            pl.BlockSpec((2, blk, CLS), lambda i: (0, i, 0)),
            pl.BlockSpec((blk, CLS), lambda i: (i, 0)),
            pl.BlockSpec((1, CLS), lambda i: (0, 0)),
        ],
        out_specs=pl.BlockSpec((blk, CLS), lambda i: (i, 0)),
        out_shape=jax.ShapeDtypeStruct((N, CLS), jnp.float32),
    )(degp, part2, g2, b2)


def kernel(x, edge_index, W1, b1, W2, b2):
    pad = EPAD - E
    padv = jnp.full((pad,), N, dtype=jnp.int32)
    srcp = jnp.concatenate([edge_index[0], padv]).reshape(ROWS, K)
    dstp = jnp.concatenate([edge_index[1], padv]).reshape(ROWS, K)
    xp = jnp.zeros((NPAD, F_IN), jnp.float32).at[:N].set(x)

    ones16 = jnp.ones((K, 16), jnp.float32)
    zeros16 = jnp.zeros((NPAD, 16), jnp.float32)
    zeros128 = jnp.zeros((NPAD, HID), jnp.float32)
    zeros64 = jnp.zeros((NPAD, CLS), jnp.float32)

    degp = _deg_kernel(dstp, ones16, zeros16)
    g1 = _tc1(degp, xp, W1)
    part1 = _spmm128(g1, srcp, dstp, zeros128)
    g2 = _tc2(degp, part1, g1, W2, b1.reshape(1, HID))
    part2 = _spmm64(g2, srcp, dstp, zeros64)
    return _tc3(degp, part2, g2, b2.reshape(1, CLS))


# speedup vs baseline: 2.2186x; 2.2173x over previous
import functools

import jax
import jax.numpy as jnp
from jax import lax
from jax.experimental import pallas as pl
from jax.experimental.pallas import tpu as pltpu
from jax.experimental.pallas import tpu_sc as plsc

N = 10000
E = 320000
F_IN = 128
HID = 128
CLS = 64

NPAD = 10240
K = 128
NTILES = 32
STEPS = 80
EPAD = NTILES * STEPS * K
ROWS = EPAD // K
S0, S1 = 80, 80
SMAX = max(S0, S1)
RPT = NPAD // 16

_MESH = dict(core_axis_name="c", subcore_axis_name="s")

ROW_BLK = 256
GRID = NPAD // ROW_BLK


def _deg_kernel(dstp, ones16, zeros16):
    @functools.partial(
        pl.kernel,
        out_type=jax.ShapeDtypeStruct((2, NPAD, 16), jnp.float32),
        mesh=plsc.VectorSubcoreMesh(**_MESH),
        compiler_params=pltpu.CompilerParams(use_tc_tiling_on_sc=False),
        scratch_types=[
            pltpu.VMEM((STEPS, K), jnp.int32),
            pltpu.VMEM((K, 16), jnp.float32),
            pltpu.VMEM_SHARED((NPAD, 16), jnp.float32),
        ],
    )
    def body(dst_hbm, ones_hbm, zeros_hbm, out_hbm, dstx, ones_v, acc):
        c = lax.axis_index("c")
        s = lax.axis_index("s")
        wid = s * 2 + c
        r0 = s * RPT
        pltpu.sync_copy(zeros_hbm.at[pl.ds(r0, RPT)], acc.at[pl.ds(r0, RPT)])
        pltpu.sync_copy(ones_hbm, ones_v)
        pltpu.sync_copy(dst_hbm.at[pl.ds(wid * STEPS, STEPS)], dstx)
        plsc.subcore_barrier()

        def step(j, carry):
            pltpu.sync_copy(ones_v, acc.at[dstx.at[j]], add=True)
            return carry

        lax.fori_loop(0, STEPS, step, 0)
        plsc.subcore_barrier()
        pltpu.sync_copy(acc.at[pl.ds(r0, RPT)], out_hbm.at[c, pl.ds(r0, RPT)])

    return body(dstp, ones16, zeros16)


def _make_spmm(width):
    @functools.partial(
        pl.kernel,
        out_type=jax.ShapeDtypeStruct((2, NPAD, width), jnp.float32),
        mesh=plsc.VectorSubcoreMesh(**_MESH),
        compiler_params=pltpu.CompilerParams(use_tc_tiling_on_sc=False),
        scratch_types=[
            pltpu.VMEM((SMAX, K), jnp.int32),
            pltpu.VMEM((SMAX, K), jnp.int32),
            pltpu.VMEM((K, width), jnp.float32),
            pltpu.VMEM_SHARED((NPAD, width), jnp.float32),
            pltpu.SemaphoreType.DMA,
        ],
    )
    def body(g_hbm, src_hbm, dst_hbm, zeros_hbm, out_hbm, srcx, dstx, buf, acc,
             sem):
        c = lax.axis_index("c")
        s = lax.axis_index("s")
        r0 = s * RPT
        base = jnp.where(c == 0, s * S0, 16 * S0 + s * S1)
        pltpu.sync_copy(zeros_hbm.at[pl.ds(r0, RPT)], acc.at[pl.ds(r0, RPT)])
        pltpu.sync_copy(src_hbm.at[pl.ds(base, SMAX)], srcx)
        pltpu.sync_copy(dst_hbm.at[pl.ds(base, SMAX)], dstx)
        plsc.subcore_barrier()

        def step(j, carry):
            pltpu.async_copy(g_hbm.at[srcx.at[j]], buf, sem).wait()
            pltpu.sync_copy(buf, acc.at[dstx.at[j]], add=True)
            return carry

        lax.fori_loop(0, S0, step, 0)
        plsc.subcore_barrier()
        pltpu.sync_copy(acc.at[pl.ds(r0, RPT)], out_hbm.at[c, pl.ds(r0, RPT)])

    return body


_spmm128 = _make_spmm(HID)
_spmm64 = _make_spmm(CLS)


def _deg_dinv(degp):
    deg = 1.0 + jnp.sum(degp[0], axis=1) + jnp.sum(degp[1], axis=1)
    dinv = lax.rsqrt(deg)
    return deg, dinv


def _tc1_body(degp_ref, x_ref, w1_ref, g1_ref):
    _, dinv = _deg_dinv(degp_ref[...])
    g1_ref[...] = jnp.dot(dinv[:, None] * x_ref[...], w1_ref[...],
                          preferred_element_type=jnp.float32)


def _tc2_body(degp_ref, part_ref, g1_ref, w2_ref, b1_ref, g2_ref):
    deg, dinv = _deg_dinv(degp_ref[...])
    p = part_ref[...]
    ssum = p[0] + p[1] + g1_ref[...]
    h1 = jnp.maximum((dinv / deg)[:, None] * ssum + b1_ref[...], 0.0)
    g2_ref[...] = dinv[:, None] * jnp.dot(h1, w2_ref[...],
                                          preferred_element_type=jnp.float32)


def _tc3_body(degp_ref, part_ref, g2_ref, b2_ref, out_ref):
    deg, dinv = _deg_dinv(degp_ref[...])
    p = part_ref[...]
    a = (dinv / deg)[:, None] * (p[0] + p[1] + g2_ref[...]) + b2_ref[...]
    m = jnp.max(a, axis=1, keepdims=True)
    ex = jnp.exp(a - m)
    out_ref[...] = (a - m) - jnp.log(jnp.sum(ex, axis=1, keepdims=True))


def _degp_spec():
    return pl.BlockSpec((2, ROW_BLK, 16), lambda i: (0, i, 0))


def _tc1(degp, xp, W1):
    return pl.pallas_call(
        _tc1_body,
        grid=(GRID,),
        in_specs=[
            _degp_spec(),
            pl.BlockSpec((ROW_BLK, F_IN), lambda i: (i, 0)),
            pl.BlockSpec((F_IN, HID), lambda i: (0, 0)),
        ],
        out_specs=pl.BlockSpec((ROW_BLK, HID), lambda i: (i, 0)),
        out_shape=jax.ShapeDtypeStruct((NPAD, HID), jnp.float32),
    )(degp, xp, W1)


def _tc2(degp, part1, g1, W2, b1):
    return pl.pallas_call(
        _tc2_body,
        grid=(GRID,),
        in_specs=[
            _degp_spec(),
            pl.BlockSpec((2, ROW_BLK, HID), lambda i: (0, i, 0)),
            pl.BlockSpec((ROW_BLK, HID), lambda i: (i, 0)),
            pl.BlockSpec((HID, CLS), lambda i: (0, 0)),
            pl.BlockSpec((1, HID), lambda i: (0, 0)),
        ],
        out_specs=pl.BlockSpec((ROW_BLK, CLS), lambda i: (i, 0)),
        out_shape=jax.ShapeDtypeStruct((NPAD, CLS), jnp.float32),
    )(degp, part1, g1, W2, b1)


def _tc3(degp, part2, g2, b2):
    blk = 400
    return pl.pallas_call(
        _tc3_body,
        grid=(N // blk,),
        in_specs=[
            pl.BlockSpec((2, blk, 16), lambda i: (0, i, 0)),
            pl.BlockSpec((2, blk, CLS), lambda i: (0, i, 0)),
            pl.BlockSpec((blk, CLS), lambda i: (i, 0)),
            pl.BlockSpec((1, CLS), lambda i: (0, 0)),
        ],
        out_specs=pl.BlockSpec((blk, CLS), lambda i: (i, 0)),
        out_shape=jax.ShapeDtypeStruct((N, CLS), jnp.float32),
    )(degp, part2, g2, b2)


def kernel(x, edge_index, W1, b1, W2, b2):
    pad = EPAD - E
    padv = N + (jnp.arange(pad, dtype=jnp.int32) % (NPAD - N))
    srcp = jnp.concatenate([edge_index[0], padv]).reshape(ROWS, K)
    dstp = jnp.concatenate([edge_index[1], padv]).reshape(ROWS, K)
    xp = jnp.zeros((NPAD, F_IN), jnp.float32).at[:N].set(x)

    ones16 = jnp.ones((K, 16), jnp.float32)
    zeros16 = jnp.zeros((NPAD, 16), jnp.float32)
    zeros128 = jnp.zeros((NPAD, HID), jnp.float32)
    zeros64 = jnp.zeros((NPAD, CLS), jnp.float32)

    degp = _deg_kernel(dstp, ones16, zeros16)
    g1 = _tc1(degp, xp, W1)
    part1 = _spmm128(g1, srcp, dstp, zeros128)
    g2 = _tc2(degp, part1, g1, W2, b1.reshape(1, HID))
    part2 = _spmm64(g2, srcp, dstp, zeros64)
    return _tc3(degp, part2, g2, b2.reshape(1, CLS))


# speedup vs baseline: 2.5651x; 1.1562x over previous
import functools

import jax
import jax.numpy as jnp
from jax import lax
from jax.experimental import pallas as pl
from jax.experimental.pallas import tpu as pltpu
from jax.experimental.pallas import tpu_sc as plsc

N = 10000
E = 320000
F_IN = 128
HID = 128
CLS = 64

NPAD = 10240
K = 128
NTILES = 32
STEPS = 80
EPAD = NTILES * STEPS * K
ROWS = EPAD // K
S0, S1 = 80, 80
SMAX = max(S0, S1)
RPT = NPAD // 16

_MESH = dict(core_axis_name="c", subcore_axis_name="s")

ROW_BLK = 256
GRID = NPAD // ROW_BLK


def _deg_kernel(dstp, ones16, zeros16):
    @functools.partial(
        pl.kernel,
        out_type=jax.ShapeDtypeStruct((2, NPAD, 16), jnp.float32),
        mesh=plsc.VectorSubcoreMesh(**_MESH),
        compiler_params=pltpu.CompilerParams(use_tc_tiling_on_sc=False),
        scratch_types=[
            pltpu.VMEM((STEPS, K), jnp.int32),
            pltpu.VMEM((K, 16), jnp.float32),
            pltpu.VMEM_SHARED((NPAD, 16), jnp.float32),
        ],
    )
    def body(dst_hbm, ones_hbm, zeros_hbm, out_hbm, dstx, ones_v, acc):
        c = lax.axis_index("c")
        s = lax.axis_index("s")
        wid = s * 2 + c
        r0 = s * RPT
        pltpu.sync_copy(zeros_hbm.at[pl.ds(r0, RPT)], acc.at[pl.ds(r0, RPT)])
        pltpu.sync_copy(ones_hbm, ones_v)
        pltpu.sync_copy(dst_hbm.at[pl.ds(wid * STEPS, STEPS)], dstx)
        plsc.subcore_barrier()

        def step(j, carry):
            pltpu.sync_copy(ones_v, acc.at[dstx.at[j]], add=True)
            return carry

        lax.fori_loop(0, STEPS, step, 0)
        plsc.subcore_barrier()
        pltpu.sync_copy(acc.at[pl.ds(r0, RPT)], out_hbm.at[c, pl.ds(r0, RPT)])

    return body(dstp, ones16, zeros16)


def _make_spmm(width):
    @functools.partial(
        pl.kernel,
        out_type=jax.ShapeDtypeStruct((2, NPAD, width), jnp.bfloat16),
        mesh=plsc.VectorSubcoreMesh(**_MESH),
        compiler_params=pltpu.CompilerParams(use_tc_tiling_on_sc=False),
        scratch_types=[
            pltpu.VMEM((SMAX, K), jnp.int32),
            pltpu.VMEM((SMAX, K), jnp.int32),
            pltpu.VMEM((K, width), jnp.bfloat16),
            pltpu.VMEM_SHARED((NPAD, width), jnp.bfloat16),
            pltpu.SemaphoreType.DMA,
        ],
    )
    def body(g_hbm, src_hbm, dst_hbm, zeros_hbm, out_hbm, srcx, dstx, buf, acc,
             sem):
        c = lax.axis_index("c")
        s = lax.axis_index("s")
        r0 = s * RPT
        base = jnp.where(c == 0, s * S0, 16 * S0 + s * S1)
        pltpu.sync_copy(zeros_hbm.at[pl.ds(r0, RPT)], acc.at[pl.ds(r0, RPT)])
        pltpu.sync_copy(src_hbm.at[pl.ds(base, SMAX)], srcx)
        pltpu.sync_copy(dst_hbm.at[pl.ds(base, SMAX)], dstx)
        plsc.subcore_barrier()

        def step(j, carry):
            pltpu.async_copy(g_hbm.at[srcx.at[j]], buf, sem).wait()
            pltpu.sync_copy(buf, acc.at[dstx.at[j]], add=True)
            return carry

        lax.fori_loop(0, S0, step, 0)
        plsc.subcore_barrier()
        pltpu.sync_copy(acc.at[pl.ds(r0, RPT)], out_hbm.at[c, pl.ds(r0, RPT)])

    return body


_spmm128 = _make_spmm(HID)
_spmm64 = _make_spmm(CLS)


def _deg_dinv(degp):
    deg = 1.0 + jnp.sum(degp[0], axis=1) + jnp.sum(degp[1], axis=1)
    dinv = lax.rsqrt(deg)
    return deg, dinv


def _tc1_body(degp_ref, x_ref, w1_ref, g1_ref):
    _, dinv = _deg_dinv(degp_ref[...])
    g1_ref[...] = jnp.dot(dinv[:, None] * x_ref[...], w1_ref[...],
                          preferred_element_type=jnp.float32
                          ).astype(jnp.bfloat16)


def _tc2_body(degp_ref, part_ref, g1_ref, w2_ref, b1_ref, g2_ref):
    deg, dinv = _deg_dinv(degp_ref[...])
    p = part_ref[...].astype(jnp.float32)
    ssum = p[0] + p[1] + g1_ref[...].astype(jnp.float32)
    h1 = jnp.maximum((dinv / deg)[:, None] * ssum + b1_ref[...], 0.0)
    g2_ref[...] = (dinv[:, None] * jnp.dot(h1, w2_ref[...],
                                           preferred_element_type=jnp.float32)
                   ).astype(jnp.bfloat16)


def _tc3_body(degp_ref, part_ref, g2_ref, b2_ref, out_ref):
    deg, dinv = _deg_dinv(degp_ref[...])
    p = part_ref[...].astype(jnp.float32)
    a = ((dinv / deg)[:, None] * (p[0] + p[1] + g2_ref[...].astype(jnp.float32))
         + b2_ref[...])
    m = jnp.max(a, axis=1, keepdims=True)
    ex = jnp.exp(a - m)
    out_ref[...] = (a - m) - jnp.log(jnp.sum(ex, axis=1, keepdims=True))


def _degp_spec():
    return pl.BlockSpec((2, ROW_BLK, 16), lambda i: (0, i, 0))


def _tc1(degp, xp, W1):
    return pl.pallas_call(
        _tc1_body,
        grid=(GRID,),
        in_specs=[
            _degp_spec(),
            pl.BlockSpec((ROW_BLK, F_IN), lambda i: (i, 0)),
            pl.BlockSpec((F_IN, HID), lambda i: (0, 0)),
        ],
        out_specs=pl.BlockSpec((ROW_BLK, HID), lambda i: (i, 0)),
        out_shape=jax.ShapeDtypeStruct((NPAD, HID), jnp.bfloat16),
    )(degp, xp, W1)


def _tc2(degp, part1, g1, W2, b1):
    return pl.pallas_call(
        _tc2_body,
        grid=(GRID,),
        in_specs=[
            _degp_spec(),
            pl.BlockSpec((2, ROW_BLK, HID), lambda i: (0, i, 0)),
            pl.BlockSpec((ROW_BLK, HID), lambda i: (i, 0)),
            pl.BlockSpec((HID, CLS), lambda i: (0, 0)),
            pl.BlockSpec((1, HID), lambda i: (0, 0)),
        ],
        out_specs=pl.BlockSpec((ROW_BLK, CLS), lambda i: (i, 0)),
        out_shape=jax.ShapeDtypeStruct((NPAD, CLS), jnp.bfloat16),
    )(degp, part1, g1, W2, b1)


def _tc3(degp, part2, g2, b2):
    blk = 400
    return pl.pallas_call(
        _tc3_body,
        grid=(N // blk,),
        in_specs=[
            pl.BlockSpec((2, blk, 16), lambda i: (0, i, 0)),
            pl.BlockSpec((2, blk, CLS), lambda i: (0, i, 0)),
            pl.BlockSpec((blk, CLS), lambda i: (i, 0)),
            pl.BlockSpec((1, CLS), lambda i: (0, 0)),
        ],
        out_specs=pl.BlockSpec((blk, CLS), lambda i: (i, 0)),
        out_shape=jax.ShapeDtypeStruct((N, CLS), jnp.float32),
    )(degp, part2, g2, b2)


def kernel(x, edge_index, W1, b1, W2, b2):
    pad = EPAD - E
    padv = N + (jnp.arange(pad, dtype=jnp.int32) % (NPAD - N))
    srcp = jnp.concatenate([edge_index[0], padv]).reshape(ROWS, K)
    dstp = jnp.concatenate([edge_index[1], padv]).reshape(ROWS, K)
    xp = jnp.zeros((NPAD, F_IN), jnp.float32).at[:N].set(x)

    ones16 = jnp.ones((K, 16), jnp.float32)
    zeros16 = jnp.zeros((NPAD, 16), jnp.float32)
    zeros128 = jnp.zeros((NPAD, HID), jnp.bfloat16)
    zeros64 = jnp.zeros((NPAD, CLS), jnp.bfloat16)

    degp = _deg_kernel(dstp, ones16, zeros16)
    g1 = _tc1(degp, xp, W1)
    part1 = _spmm128(g1, srcp, dstp, zeros128)
    g2 = _tc2(degp, part1, g1, W2, b1.reshape(1, HID))
    part2 = _spmm64(g2, srcp, dstp, zeros64)
    return _tc3(degp, part2, g2, b2.reshape(1, CLS))


# speedup vs baseline: 3.2591x; 1.2705x over previous
import functools

import jax
import jax.numpy as jnp
from jax import lax
from jax.experimental import pallas as pl
from jax.experimental.pallas import tpu as pltpu
from jax.experimental.pallas import tpu_sc as plsc

N = 10000
E = 320000
F_IN = 128
HID = 128
CLS = 64

NPAD = 10240
K = 128
NTILES = 32
STEPS = 80
EPAD = NTILES * STEPS * K
ROWS = EPAD // K
S0, S1 = 80, 80
SMAX = max(S0, S1)
RPT = NPAD // 16

_MESH = dict(core_axis_name="c", subcore_axis_name="s")

ROW_BLK = 256
GRID = NPAD // ROW_BLK


def _deg_kernel(dstp, ones16, zeros16):
    @functools.partial(
        pl.kernel,
        out_type=jax.ShapeDtypeStruct((2, NPAD, 16), jnp.float32),
        mesh=plsc.VectorSubcoreMesh(**_MESH),
        compiler_params=pltpu.CompilerParams(use_tc_tiling_on_sc=False),
        scratch_types=[
            pltpu.VMEM((STEPS, K), jnp.int32),
            pltpu.VMEM((K, 16), jnp.float32),
            pltpu.VMEM_SHARED((NPAD, 16), jnp.float32),
        ],
    )
    def body(dst_hbm, ones_hbm, zeros_hbm, out_hbm, dstx, ones_v, acc):
        c = lax.axis_index("c")
        s = lax.axis_index("s")
        wid = s * 2 + c
        r0 = s * RPT
        pltpu.sync_copy(zeros_hbm.at[pl.ds(r0, RPT)], acc.at[pl.ds(r0, RPT)])
        pltpu.sync_copy(ones_hbm, ones_v)
        pltpu.sync_copy(dst_hbm.at[pl.ds(wid * STEPS, STEPS)], dstx)
        plsc.subcore_barrier()

        def step(j, carry):
            pltpu.sync_copy(ones_v, acc.at[dstx.at[j]], add=True)
            return carry

        lax.fori_loop(0, STEPS, step, 0)
        plsc.subcore_barrier()
        pltpu.sync_copy(acc.at[pl.ds(r0, RPT)], out_hbm.at[c, pl.ds(r0, RPT)])

    return body(dstp, ones16, zeros16)


def _make_spmm(width):
    @functools.partial(
        pl.kernel,
        out_type=jax.ShapeDtypeStruct((2, NPAD, width), jnp.bfloat16),
        mesh=plsc.VectorSubcoreMesh(**_MESH),
        compiler_params=pltpu.CompilerParams(use_tc_tiling_on_sc=False),
        scratch_types=[
            pltpu.VMEM((SMAX, K), jnp.int32),
            pltpu.VMEM((SMAX, K), jnp.int32),
            pltpu.VMEM((2, K, width), jnp.bfloat16),
            pltpu.VMEM_SHARED((NPAD, width), jnp.bfloat16),
            pltpu.SemaphoreType.DMA,
            pltpu.SemaphoreType.DMA,
        ],
    )
    def body(g_hbm, src_hbm, dst_hbm, zeros_hbm, out_hbm, srcx, dstx, buf, acc,
             sem0, sem1):
        c = lax.axis_index("c")
        s = lax.axis_index("s")
        r0 = s * RPT
        base = jnp.where(c == 0, s * S0, 16 * S0 + s * S1)
        pltpu.sync_copy(zeros_hbm.at[pl.ds(r0, RPT)], acc.at[pl.ds(r0, RPT)])
        pltpu.sync_copy(src_hbm.at[pl.ds(base, SMAX)], srcx)
        pltpu.sync_copy(dst_hbm.at[pl.ds(base, SMAX)], dstx)
        plsc.subcore_barrier()

        sems = (sem0, sem1)
        pltpu.async_copy(g_hbm.at[srcx.at[0]], buf.at[0], sems[0])

        def step(jo, carry):
            for b in range(2):
                j = 2 * jo + b
                slot = b
                nslot = 1 - b

                @pl.when(j + 1 < S0)
                def _():
                    pltpu.async_copy(g_hbm.at[srcx.at[j + 1]], buf.at[nslot],
                                     sems[nslot])

                pltpu.make_async_copy(g_hbm.at[srcx.at[j]], buf.at[slot],
                                      sems[slot]).wait()
                pltpu.sync_copy(buf.at[slot], acc.at[dstx.at[j]], add=True)
            return carry

        lax.fori_loop(0, S0 // 2, step, 0)
        plsc.subcore_barrier()
        pltpu.sync_copy(acc.at[pl.ds(r0, RPT)], out_hbm.at[c, pl.ds(r0, RPT)])

    return body


_spmm128 = _make_spmm(HID)
_spmm64 = _make_spmm(CLS)


def _deg_dinv(degp):
    deg = 1.0 + jnp.sum(degp[0], axis=1) + jnp.sum(degp[1], axis=1)
    dinv = lax.rsqrt(deg)
    return deg, dinv


def _tc1_body(degp_ref, x_ref, w1_ref, g1_ref):
    _, dinv = _deg_dinv(degp_ref[...])
    g1_ref[...] = jnp.dot(dinv[:, None] * x_ref[...], w1_ref[...],
                          preferred_element_type=jnp.float32
                          ).astype(jnp.bfloat16)


def _tc2_body(degp_ref, part_ref, g1_ref, w2_ref, b1_ref, g2_ref):
    deg, dinv = _deg_dinv(degp_ref[...])
    p = part_ref[...].astype(jnp.float32)
    ssum = p[0] + p[1] + g1_ref[...].astype(jnp.float32)
    h1 = jnp.maximum((dinv / deg)[:, None] * ssum + b1_ref[...], 0.0)
    g2_ref[...] = (dinv[:, None] * jnp.dot(h1, w2_ref[...],
                                           preferred_element_type=jnp.float32)
                   ).astype(jnp.bfloat16)


def _tc3_body(degp_ref, part_ref, g2_ref, b2_ref, out_ref):
    deg, dinv = _deg_dinv(degp_ref[...])
    p = part_ref[...].astype(jnp.float32)
    a = ((dinv / deg)[:, None] * (p[0] + p[1] + g2_ref[...].astype(jnp.float32))
         + b2_ref[...])
    m = jnp.max(a, axis=1, keepdims=True)
    ex = jnp.exp(a - m)
    out_ref[...] = (a - m) - jnp.log(jnp.sum(ex, axis=1, keepdims=True))


def _degp_spec():
    return pl.BlockSpec((2, ROW_BLK, 16), lambda i: (0, i, 0))


def _tc1(degp, xp, W1):
    return pl.pallas_call(
        _tc1_body,
        grid=(GRID,),
        in_specs=[
            _degp_spec(),
            pl.BlockSpec((ROW_BLK, F_IN), lambda i: (i, 0)),
            pl.BlockSpec((F_IN, HID), lambda i: (0, 0)),
        ],
        out_specs=pl.BlockSpec((ROW_BLK, HID), lambda i: (i, 0)),
        out_shape=jax.ShapeDtypeStruct((NPAD, HID), jnp.bfloat16),
    )(degp, xp, W1)


def _tc2(degp, part1, g1, W2, b1):
    return pl.pallas_call(
        _tc2_body,
        grid=(GRID,),
        in_specs=[
            _degp_spec(),
            pl.BlockSpec((2, ROW_BLK, HID), lambda i: (0, i, 0)),
            pl.BlockSpec((ROW_BLK, HID), lambda i: (i, 0)),
            pl.BlockSpec((HID, CLS), lambda i: (0, 0)),
            pl.BlockSpec((1, HID), lambda i: (0, 0)),
        ],
        out_specs=pl.BlockSpec((ROW_BLK, CLS), lambda i: (i, 0)),
        out_shape=jax.ShapeDtypeStruct((NPAD, CLS), jnp.bfloat16),
    )(degp, part1, g1, W2, b1)


def _tc3(degp, part2, g2, b2):
    blk = 400
    return pl.pallas_call(
        _tc3_body,
        grid=(N // blk,),
        in_specs=[
            pl.BlockSpec((2, blk, 16), lambda i: (0, i, 0)),
            pl.BlockSpec((2, blk, CLS), lambda i: (0, i, 0)),
            pl.BlockSpec((blk, CLS), lambda i: (i, 0)),
            pl.BlockSpec((1, CLS), lambda i: (0, 0)),
        ],
        out_specs=pl.BlockSpec((blk, CLS), lambda i: (i, 0)),
        out_shape=jax.ShapeDtypeStruct((N, CLS), jnp.float32),
    )(degp, part2, g2, b2)


def kernel(x, edge_index, W1, b1, W2, b2):
    pad = EPAD - E
    padv = N + (jnp.arange(pad, dtype=jnp.int32) % (NPAD - N))
    srcp = jnp.concatenate([edge_index[0], padv]).reshape(ROWS, K)
    dstp = jnp.concatenate([edge_index[1], padv]).reshape(ROWS, K)
    xp = jnp.zeros((NPAD, F_IN), jnp.float32).at[:N].set(x)

    ones16 = jnp.ones((K, 16), jnp.float32)
    zeros16 = jnp.zeros((NPAD, 16), jnp.float32)
    zeros128 = jnp.zeros((NPAD, HID), jnp.bfloat16)
    zeros64 = jnp.zeros((NPAD, CLS), jnp.bfloat16)

    degp = _deg_kernel(dstp, ones16, zeros16)
    g1 = _tc1(degp, xp, W1)
    part1 = _spmm128(g1, srcp, dstp, zeros128)
    g2 = _tc2(degp, part1, g1, W2, b1.reshape(1, HID))
    part2 = _spmm64(g2, srcp, dstp, zeros64)
    return _tc3(degp, part2, g2, b2.reshape(1, CLS))


# speedup vs baseline: 3.6369x; 1.1159x over previous
import functools

import jax
import jax.numpy as jnp
from jax import lax
from jax.experimental import pallas as pl
from jax.experimental.pallas import tpu as pltpu
from jax.experimental.pallas import tpu_sc as plsc

N = 10000
E = 320000
F_IN = 128
HID = 128
CLS = 64

NPAD = 10240
K = 128
NTILES = 32
STEPS = 80
EPAD = NTILES * STEPS * K
ROWS = EPAD // K
S0, S1 = 80, 80
SMAX = max(S0, S1)
RPT = NPAD // 16

_MESH = dict(core_axis_name="c", subcore_axis_name="s")

ROW_BLK = 256
GRID = NPAD // ROW_BLK


def _deg_kernel(dstp, ones16, zeros16):
    @functools.partial(
        pl.kernel,
        out_type=jax.ShapeDtypeStruct((2, NPAD, 16), jnp.float32),
        mesh=plsc.VectorSubcoreMesh(**_MESH),
        compiler_params=pltpu.CompilerParams(use_tc_tiling_on_sc=False),
        scratch_types=[
            pltpu.VMEM((STEPS, K), jnp.int32),
            pltpu.VMEM((K, 16), jnp.float32),
            pltpu.VMEM_SHARED((NPAD, 16), jnp.float32),
        ],
    )
    def body(dst_hbm, ones_hbm, zeros_hbm, out_hbm, dstx, ones_v, acc):
        c = lax.axis_index("c")
        s = lax.axis_index("s")
        wid = s * 2 + c
        r0 = s * RPT
        pltpu.sync_copy(zeros_hbm.at[pl.ds(r0, RPT)], acc.at[pl.ds(r0, RPT)])
        pltpu.sync_copy(ones_hbm, ones_v)
        pltpu.sync_copy(dst_hbm.at[pl.ds(wid * STEPS, STEPS)], dstx)
        plsc.subcore_barrier()

        def step(j, carry):
            pltpu.sync_copy(ones_v, acc.at[dstx.at[j]], add=True)
            return carry

        lax.fori_loop(0, STEPS, step, 0)
        plsc.subcore_barrier()
        pltpu.sync_copy(acc.at[pl.ds(r0, RPT)], out_hbm.at[c, pl.ds(r0, RPT)])

    return body(dstp, ones16, zeros16)


def _make_spmm(width):
    @functools.partial(
        pl.kernel,
        out_type=jax.ShapeDtypeStruct((2, NPAD, width), jnp.bfloat16),
        mesh=plsc.VectorSubcoreMesh(**_MESH),
        compiler_params=pltpu.CompilerParams(use_tc_tiling_on_sc=False),
        scratch_types=[
            pltpu.VMEM((SMAX, K), jnp.int32),
            pltpu.VMEM((SMAX, K), jnp.int32),
            pltpu.VMEM((4, K, width), jnp.bfloat16),
            pltpu.VMEM_SHARED((NPAD, width), jnp.bfloat16),
            pltpu.SemaphoreType.DMA,
            pltpu.SemaphoreType.DMA,
            pltpu.SemaphoreType.DMA,
            pltpu.SemaphoreType.DMA,
        ],
    )
    def body(g_hbm, src_hbm, dst_hbm, zeros_hbm, out_hbm, srcx, dstx, buf, acc,
             sem0, sem1, sem2, sem3):
        c = lax.axis_index("c")
        s = lax.axis_index("s")
        r0 = s * RPT
        base = jnp.where(c == 0, s * S0, 16 * S0 + s * S1)
        pltpu.sync_copy(zeros_hbm.at[pl.ds(r0, RPT)], acc.at[pl.ds(r0, RPT)])
        pltpu.sync_copy(src_hbm.at[pl.ds(base, SMAX)], srcx)
        pltpu.sync_copy(dst_hbm.at[pl.ds(base, SMAX)], dstx)
        plsc.subcore_barrier()

        sems = (sem0, sem1, sem2, sem3)
        for p in range(3):
            pltpu.async_copy(g_hbm.at[srcx.at[p]], buf.at[p], sems[p])

        def step(jo, carry):
            for b in range(4):
                j = 4 * jo + b
                nslot = (b + 3) % 4

                @pl.when(j + 3 < S0)
                def _():
                    pltpu.async_copy(g_hbm.at[srcx.at[j + 3]], buf.at[nslot],
                                     sems[nslot])

                pltpu.make_async_copy(g_hbm.at[srcx.at[j]], buf.at[b],
                                      sems[b]).wait()
                pltpu.sync_copy(buf.at[b], acc.at[dstx.at[j]], add=True)
            return carry

        lax.fori_loop(0, S0 // 4, step, 0)
        plsc.subcore_barrier()
        pltpu.sync_copy(acc.at[pl.ds(r0, RPT)], out_hbm.at[c, pl.ds(r0, RPT)])

    return body


_spmm128 = _make_spmm(HID)
_spmm64 = _make_spmm(CLS)


def _deg_dinv(degp):
    deg = 1.0 + jnp.sum(degp[0], axis=1) + jnp.sum(degp[1], axis=1)
    dinv = lax.rsqrt(deg)
    return deg, dinv


def _tc1_body(degp_ref, x_ref, w1_ref, g1_ref):
    _, dinv = _deg_dinv(degp_ref[...])
    g1_ref[...] = jnp.dot(dinv[:, None] * x_ref[...], w1_ref[...],
                          preferred_element_type=jnp.float32
                          ).astype(jnp.bfloat16)


def _tc2_body(degp_ref, part_ref, g1_ref, w2_ref, b1_ref, g2_ref):
    deg, dinv = _deg_dinv(degp_ref[...])
    p = part_ref[...].astype(jnp.float32)
    ssum = p[0] + p[1] + g1_ref[...].astype(jnp.float32)
    h1 = jnp.maximum((dinv / deg)[:, None] * ssum + b1_ref[...], 0.0)
    g2_ref[...] = (dinv[:, None] * jnp.dot(h1, w2_ref[...],
                                           preferred_element_type=jnp.float32)
                   ).astype(jnp.bfloat16)


def _tc3_body(degp_ref, part_ref, g2_ref, b2_ref, out_ref):
    deg, dinv = _deg_dinv(degp_ref[...])
    p = part_ref[...].astype(jnp.float32)
    a = ((dinv / deg)[:, None] * (p[0] + p[1] + g2_ref[...].astype(jnp.float32))
         + b2_ref[...])
    m = jnp.max(a, axis=1, keepdims=True)
    ex = jnp.exp(a - m)
    out_ref[...] = (a - m) - jnp.log(jnp.sum(ex, axis=1, keepdims=True))


def _degp_spec():
    return pl.BlockSpec((2, ROW_BLK, 16), lambda i: (0, i, 0))


def _tc1(degp, xp, W1):
    return pl.pallas_call(
        _tc1_body,
        grid=(GRID,),
        in_specs=[
            _degp_spec(),
            pl.BlockSpec((ROW_BLK, F_IN), lambda i: (i, 0)),
            pl.BlockSpec((F_IN, HID), lambda i: (0, 0)),
        ],
        out_specs=pl.BlockSpec((ROW_BLK, HID), lambda i: (i, 0)),
        out_shape=jax.ShapeDtypeStruct((NPAD, HID), jnp.bfloat16),
    )(degp, xp, W1)


def _tc2(degp, part1, g1, W2, b1):
    return pl.pallas_call(
        _tc2_body,
        grid=(GRID,),
        in_specs=[
            _degp_spec(),
            pl.BlockSpec((2, ROW_BLK, HID), lambda i: (0, i, 0)),
            pl.BlockSpec((ROW_BLK, HID), lambda i: (i, 0)),
            pl.BlockSpec((HID, CLS), lambda i: (0, 0)),
            pl.BlockSpec((1, HID), lambda i: (0, 0)),
        ],
        out_specs=pl.BlockSpec((ROW_BLK, CLS), lambda i: (i, 0)),
        out_shape=jax.ShapeDtypeStruct((NPAD, CLS), jnp.bfloat16),
    )(degp, part1, g1, W2, b1)


def _tc3(degp, part2, g2, b2):
    blk = 400
    return pl.pallas_call(
        _tc3_body,
        grid=(N // blk,),
        in_specs=[
            pl.BlockSpec((2, blk, 16), lambda i: (0, i, 0)),
            pl.BlockSpec((2, blk, CLS), lambda i: (0, i, 0)),
            pl.BlockSpec((blk, CLS), lambda i: (i, 0)),
            pl.BlockSpec((1, CLS), lambda i: (0, 0)),
        ],
        out_specs=pl.BlockSpec((blk, CLS), lambda i: (i, 0)),
        out_shape=jax.ShapeDtypeStruct((N, CLS), jnp.float32),
    )(degp, part2, g2, b2)


def kernel(x, edge_index, W1, b1, W2, b2):
    pad = EPAD - E
    padv = N + (jnp.arange(pad, dtype=jnp.int32) % (NPAD - N))
    srcp = jnp.concatenate([edge_index[0], padv]).reshape(ROWS, K)
    dstp = jnp.concatenate([edge_index[1], padv]).reshape(ROWS, K)
    xp = jnp.zeros((NPAD, F_IN), jnp.float32).at[:N].set(x)

    ones16 = jnp.ones((K, 16), jnp.float32)
    zeros16 = jnp.zeros((NPAD, 16), jnp.float32)
    zeros128 = jnp.zeros((NPAD, HID), jnp.bfloat16)
    zeros64 = jnp.zeros((NPAD, CLS), jnp.bfloat16)

    degp = _deg_kernel(dstp, ones16, zeros16)
    g1 = _tc1(degp, xp, W1)
    part1 = _spmm128(g1, srcp, dstp, zeros128)
    g2 = _tc2(degp, part1, g1, W2, b1.reshape(1, HID))
    part2 = _spmm64(g2, srcp, dstp, zeros64)
    return _tc3(degp, part2, g2, b2.reshape(1, CLS))


# speedup vs baseline: 3.8920x; 1.0701x over previous
import functools

import jax
import jax.numpy as jnp
from jax import lax
from jax.experimental import pallas as pl
from jax.experimental.pallas import tpu as pltpu
from jax.experimental.pallas import tpu_sc as plsc

N = 10000
E = 320000
F_IN = 128
HID = 128
CLS = 64

NPAD = 10240
K = 128
NTILES = 32
STEPS = 80
EPAD = NTILES * STEPS * K
ROWS = EPAD // K
S0, S1 = 80, 80
SMAX = max(S0, S1)
RPT = NPAD // 16

_MESH = dict(core_axis_name="c", subcore_axis_name="s")

ROW_BLK = 256
GRID = NPAD // ROW_BLK


def _deg_kernel(dstp, ones16, zeros16):
    @functools.partial(
        pl.kernel,
        out_type=jax.ShapeDtypeStruct((2, NPAD, 16), jnp.float32),
        mesh=plsc.VectorSubcoreMesh(**_MESH),
        compiler_params=pltpu.CompilerParams(use_tc_tiling_on_sc=False),
        scratch_types=[
            pltpu.VMEM((STEPS, K), jnp.int32),
            pltpu.VMEM((K, 16), jnp.float32),
            pltpu.VMEM_SHARED((NPAD, 16), jnp.float32),
        ],
    )
    def body(dst_hbm, ones_hbm, zeros_hbm, out_hbm, dstx, ones_v, acc):
        c = lax.axis_index("c")
        s = lax.axis_index("s")
        wid = s * 2 + c
        r0 = s * RPT
        pltpu.sync_copy(zeros_hbm.at[pl.ds(r0, RPT)], acc.at[pl.ds(r0, RPT)])
        pltpu.sync_copy(ones_hbm, ones_v)
        pltpu.sync_copy(dst_hbm.at[pl.ds(wid * STEPS, STEPS)], dstx)
        plsc.subcore_barrier()

        def step(j, carry):
            pltpu.sync_copy(ones_v, acc.at[dstx.at[j]], add=True)
            return carry

        lax.fori_loop(0, STEPS, step, 0)
        plsc.subcore_barrier()
        pltpu.sync_copy(acc.at[pl.ds(r0, RPT)], out_hbm.at[c, pl.ds(r0, RPT)])

    return body(dstp, ones16, zeros16)


def _make_spmm(width):
    @functools.partial(
        pl.kernel,
        out_type=jax.ShapeDtypeStruct((2, NPAD, width), jnp.bfloat16),
        mesh=plsc.VectorSubcoreMesh(**_MESH),
        compiler_params=pltpu.CompilerParams(use_tc_tiling_on_sc=False),
        scratch_types=[
            pltpu.VMEM((SMAX, K), jnp.int32),
            pltpu.VMEM((SMAX, K), jnp.int32),
            pltpu.VMEM((4, K, width), jnp.bfloat16),
            pltpu.VMEM_SHARED((NPAD, width), jnp.bfloat16),
            pltpu.SemaphoreType.DMA,
            pltpu.SemaphoreType.DMA,
            pltpu.SemaphoreType.DMA,
            pltpu.SemaphoreType.DMA,
        ],
    )
    def body(g_hbm, src_hbm, dst_hbm, zeros_hbm, out_hbm, srcx, dstx, buf, acc,
             sem0, sem1, sem2, sem3):
        c = lax.axis_index("c")
        s = lax.axis_index("s")
        r0 = s * RPT
        base = jnp.where(c == 0, s * S0, 16 * S0 + s * S1)
        pltpu.sync_copy(zeros_hbm.at[pl.ds(r0, RPT)], acc.at[pl.ds(r0, RPT)])
        pltpu.sync_copy(src_hbm.at[pl.ds(base, SMAX)], srcx)
        pltpu.sync_copy(dst_hbm.at[pl.ds(base, SMAX)], dstx)
        plsc.subcore_barrier()

        sems = (sem0, sem1, sem2, sem3)
        for p in range(3):
            pltpu.async_copy(g_hbm.at[srcx.at[p]], buf.at[p], sems[p])

        def step(jo, carry):
            for b in range(4):
                j = 4 * jo + b
                nslot = (b + 3) % 4

                @pl.when(j + 3 < S0)
                def _():
                    pltpu.async_copy(g_hbm.at[srcx.at[j + 3]], buf.at[nslot],
                                     sems[nslot])

                pltpu.make_async_copy(g_hbm.at[srcx.at[j]], buf.at[b],
                                      sems[b]).wait()
                pltpu.sync_copy(buf.at[b], acc.at[dstx.at[j]], add=True)
            return carry

        lax.fori_loop(0, S0 // 4, step, 0)
        plsc.subcore_barrier()
        pltpu.sync_copy(acc.at[pl.ds(r0, RPT)], out_hbm.at[c, pl.ds(r0, RPT)])

    return body


_spmm128 = _make_spmm(HID)
_spmm64 = _make_spmm(CLS)


def _deg_dinv(degp):
    deg = 1.0 + jnp.sum(degp[0], axis=1) + jnp.sum(degp[1], axis=1)
    dinv = lax.rsqrt(deg)
    return deg, dinv


def _tc1_body(degp_ref, x_ref, w1_ref, g1_ref, dinv_ref, sca_ref):
    deg, dinv = _deg_dinv(degp_ref[...])
    g1_ref[...] = jnp.dot(dinv[:, None] * x_ref[...], w1_ref[...],
                          preferred_element_type=jnp.float32
                          ).astype(jnp.bfloat16)
    dinv_ref[...] = dinv[:, None]
    sca_ref[...] = (dinv / deg)[:, None]


def _tc2_body(dinv_ref, sca_ref, part_ref, g1_ref, w2_ref, b1_ref, g2_ref):
    p = part_ref[...].astype(jnp.float32)
    ssum = p[0] + p[1] + g1_ref[...].astype(jnp.float32)
    h1 = jnp.maximum(sca_ref[...] * ssum + b1_ref[...], 0.0)
    g2_ref[...] = (dinv_ref[...] * jnp.dot(h1, w2_ref[...],
                                           preferred_element_type=jnp.float32)
                   ).astype(jnp.bfloat16)


def _tc3_body(sca_ref, part_ref, g2_ref, b2_ref, out_ref):
    p = part_ref[...].astype(jnp.float32)
    a = (sca_ref[...] * (p[0] + p[1] + g2_ref[...].astype(jnp.float32))
         + b2_ref[...])
    m = jnp.max(a, axis=1, keepdims=True)
    ex = jnp.exp(a - m)
    out_ref[...] = (a - m) - jnp.log(jnp.sum(ex, axis=1, keepdims=True))


BLK = 400
TGRID = N // BLK


def _tc1(degp, x, W1):
    return pl.pallas_call(
        _tc1_body,
        grid=(TGRID,),
        in_specs=[
            pl.BlockSpec((2, BLK, 16), lambda i: (0, i, 0)),
            pl.BlockSpec((BLK, F_IN), lambda i: (i, 0)),
            pl.BlockSpec((F_IN, HID), lambda i: (0, 0)),
        ],
        out_specs=[
            pl.BlockSpec((BLK, HID), lambda i: (i, 0)),
            pl.BlockSpec((BLK, 1), lambda i: (i, 0)),
            pl.BlockSpec((BLK, 1), lambda i: (i, 0)),
        ],
        out_shape=[
            jax.ShapeDtypeStruct((NPAD, HID), jnp.bfloat16),
            jax.ShapeDtypeStruct((N, 1), jnp.float32),
            jax.ShapeDtypeStruct((N, 1), jnp.float32),
        ],
    )(degp, x, W1)


def _tc2(dinv, sca, part1, g1, W2, b1):
    return pl.pallas_call(
        _tc2_body,
        grid=(TGRID,),
        in_specs=[
            pl.BlockSpec((BLK, 1), lambda i: (i, 0)),
            pl.BlockSpec((BLK, 1), lambda i: (i, 0)),
            pl.BlockSpec((2, BLK, HID), lambda i: (0, i, 0)),
            pl.BlockSpec((BLK, HID), lambda i: (i, 0)),
            pl.BlockSpec((HID, CLS), lambda i: (0, 0)),
            pl.BlockSpec((1, HID), lambda i: (0, 0)),
        ],
        out_specs=pl.BlockSpec((BLK, CLS), lambda i: (i, 0)),
        out_shape=jax.ShapeDtypeStruct((NPAD, CLS), jnp.bfloat16),
    )(dinv, sca, part1, g1, W2, b1)


def _tc3(sca, part2, g2, b2):
    return pl.pallas_call(
        _tc3_body,
        grid=(TGRID,),
        in_specs=[
            pl.BlockSpec((BLK, 1), lambda i: (i, 0)),
            pl.BlockSpec((2, BLK, CLS), lambda i: (0, i, 0)),
            pl.BlockSpec((BLK, CLS), lambda i: (i, 0)),
            pl.BlockSpec((1, CLS), lambda i: (0, 0)),
        ],
        out_specs=pl.BlockSpec((BLK, CLS), lambda i: (i, 0)),
        out_shape=jax.ShapeDtypeStruct((N, CLS), jnp.float32),
    )(sca, part2, g2, b2)


def kernel(x, edge_index, W1, b1, W2, b2):
    pad = EPAD - E
    padv = N + (jnp.arange(pad, dtype=jnp.int32) % (NPAD - N))
    srcp = jnp.concatenate([edge_index[0], padv]).reshape(ROWS, K)
    dstp = jnp.concatenate([edge_index[1], padv]).reshape(ROWS, K)

    ones16 = jnp.ones((K, 16), jnp.float32)
    zeros16 = jnp.zeros((NPAD, 16), jnp.float32)
    zeros128 = jnp.zeros((NPAD, HID), jnp.bfloat16)
    zeros64 = jnp.zeros((NPAD, CLS), jnp.bfloat16)

    degp = _deg_kernel(dstp, ones16, zeros16)
    g1, dinv, sca = _tc1(degp, x, W1)
    part1 = _spmm128(g1, srcp, dstp, zeros128)
    g2 = _tc2(dinv, sca, part1, g1, W2, b1.reshape(1, HID))
    part2 = _spmm64(g2, srcp, dstp, zeros64)
    return _tc3(sca, part2, g2, b2.reshape(1, CLS))


# speedup vs baseline: 3.9176x; 1.0066x over previous
import functools

import jax
import jax.numpy as jnp
from jax import lax
from jax.experimental import pallas as pl
from jax.experimental.pallas import tpu as pltpu
from jax.experimental.pallas import tpu_sc as plsc

N = 10000
E = 320000
F_IN = 128
HID = 128
CLS = 64

NPAD = 10240
K = 128
NTILES = 32
STEPS = 80
EPAD = NTILES * STEPS * K
ROWS = EPAD // K
S0, S1 = 80, 80
SMAX = max(S0, S1)
RPT = NPAD // 16

_MESH = dict(core_axis_name="c", subcore_axis_name="s")

ROW_BLK = 256
GRID = NPAD // ROW_BLK


def _deg_kernel(dstp, ones16, zeros16):
    @functools.partial(
        pl.kernel,
        out_type=jax.ShapeDtypeStruct((2, NPAD, 16), jnp.float32),
        mesh=plsc.VectorSubcoreMesh(**_MESH),
        compiler_params=pltpu.CompilerParams(use_tc_tiling_on_sc=False),
        scratch_types=[
            pltpu.VMEM((STEPS, K), jnp.int32),
            pltpu.VMEM((K, 16), jnp.float32),
            pltpu.VMEM_SHARED((NPAD, 16), jnp.float32),
        ],
    )
    def body(dst_hbm, ones_hbm, zeros_hbm, out_hbm, dstx, ones_v, acc):
        c = lax.axis_index("c")
        s = lax.axis_index("s")
        wid = s * 2 + c
        r0 = s * RPT
        pltpu.sync_copy(zeros_hbm.at[pl.ds(r0, RPT)], acc.at[pl.ds(r0, RPT)])
        pltpu.sync_copy(ones_hbm, ones_v)
        pltpu.sync_copy(dst_hbm.at[pl.ds(wid * STEPS, STEPS)], dstx)
        plsc.subcore_barrier()

        def step(j, carry):
            pltpu.sync_copy(ones_v, acc.at[dstx.at[j]], add=True)
            return carry

        lax.fori_loop(0, STEPS, step, 0)
        plsc.subcore_barrier()
        pltpu.sync_copy(acc.at[pl.ds(r0, RPT)], out_hbm.at[c, pl.ds(r0, RPT)])

    return body(dstp, ones16, zeros16)


def _make_spmm(width):
    @functools.partial(
        pl.kernel,
        out_type=jax.ShapeDtypeStruct((2, NPAD, width), jnp.bfloat16),
        mesh=plsc.VectorSubcoreMesh(**_MESH),
        compiler_params=pltpu.CompilerParams(use_tc_tiling_on_sc=False),
        scratch_types=[
            pltpu.VMEM((SMAX, K), jnp.int32),
            pltpu.VMEM((SMAX, K), jnp.int32),
            pltpu.VMEM((8, K, width), jnp.bfloat16),
            pltpu.VMEM_SHARED((NPAD, width), jnp.bfloat16),
        ] + [pltpu.SemaphoreType.DMA] * 8,
    )
    def body(g_hbm, src_hbm, dst_hbm, zeros_hbm, out_hbm, srcx, dstx, buf, acc,
             *sems):
        c = lax.axis_index("c")
        s = lax.axis_index("s")
        r0 = s * RPT
        base = jnp.where(c == 0, s * S0, 16 * S0 + s * S1)
        pltpu.sync_copy(zeros_hbm.at[pl.ds(r0, RPT)], acc.at[pl.ds(r0, RPT)])
        pltpu.sync_copy(src_hbm.at[pl.ds(base, SMAX)], srcx)
        pltpu.sync_copy(dst_hbm.at[pl.ds(base, SMAX)], dstx)
        plsc.subcore_barrier()

        NBUF = 8
        for p in range(NBUF - 1):
            pltpu.async_copy(g_hbm.at[srcx.at[p]], buf.at[p], sems[p])

        def step(jo, carry):
            for b in range(NBUF):
                j = NBUF * jo + b
                nslot = (b + NBUF - 1) % NBUF

                @pl.when(j + NBUF - 1 < S0)
                def _():
                    pltpu.async_copy(g_hbm.at[srcx.at[j + NBUF - 1]],
                                     buf.at[nslot], sems[nslot])

                pltpu.make_async_copy(g_hbm.at[srcx.at[j]], buf.at[b],
                                      sems[b]).wait()
                pltpu.sync_copy(buf.at[b], acc.at[dstx.at[j]], add=True)
            return carry

        lax.fori_loop(0, S0 // NBUF, step, 0)
        plsc.subcore_barrier()
        pltpu.sync_copy(acc.at[pl.ds(r0, RPT)], out_hbm.at[c, pl.ds(r0, RPT)])

    return body


_spmm128 = _make_spmm(HID)
_spmm64 = _make_spmm(CLS)


def _deg_dinv(degp):
    deg = 1.0 + jnp.sum(degp[0], axis=1) + jnp.sum(degp[1], axis=1)
    dinv = lax.rsqrt(deg)
    return deg, dinv


def _tc1_body(degp_ref, x_ref, w1_ref, g1_ref, dinv_ref, sca_ref):
    deg, dinv = _deg_dinv(degp_ref[...])
    g1_ref[...] = jnp.dot(dinv[:, None] * x_ref[...], w1_ref[...],
                          preferred_element_type=jnp.float32
                          ).astype(jnp.bfloat16)
    dinv_ref[...] = dinv[:, None]
    sca_ref[...] = (dinv / deg)[:, None]


def _tc2_body(dinv_ref, sca_ref, part_ref, g1_ref, w2_ref, b1_ref, g2_ref):
    p = part_ref[...].astype(jnp.float32)
    ssum = p[0] + p[1] + g1_ref[...].astype(jnp.float32)
    h1 = jnp.maximum(sca_ref[...] * ssum + b1_ref[...], 0.0)
    g2_ref[...] = (dinv_ref[...] * jnp.dot(h1, w2_ref[...],
                                           preferred_element_type=jnp.float32)
                   ).astype(jnp.bfloat16)


def _tc3_body(sca_ref, part_ref, g2_ref, b2_ref, out_ref):
    p = part_ref[...].astype(jnp.float32)
    a = (sca_ref[...] * (p[0] + p[1] + g2_ref[...].astype(jnp.float32))
         + b2_ref[...])
    m = jnp.max(a, axis=1, keepdims=True)
    ex = jnp.exp(a - m)
    out_ref[...] = (a - m) - jnp.log(jnp.sum(ex, axis=1, keepdims=True))


BLK = 400
TGRID = N // BLK


def _tc1(degp, x, W1):
    return pl.pallas_call(
        _tc1_body,
        grid=(TGRID,),
        in_specs=[
            pl.BlockSpec((2, BLK, 16), lambda i: (0, i, 0)),
            pl.BlockSpec((BLK, F_IN), lambda i: (i, 0)),
            pl.BlockSpec((F_IN, HID), lambda i: (0, 0)),
        ],
        out_specs=[
            pl.BlockSpec((BLK, HID), lambda i: (i, 0)),
            pl.BlockSpec((BLK, 1), lambda i: (i, 0)),
            pl.BlockSpec((BLK, 1), lambda i: (i, 0)),
        ],
        out_shape=[
            jax.ShapeDtypeStruct((NPAD, HID), jnp.bfloat16),
            jax.ShapeDtypeStruct((N, 1), jnp.float32),
            jax.ShapeDtypeStruct((N, 1), jnp.float32),
        ],
    )(degp, x, W1)


def _tc2(dinv, sca, part1, g1, W2, b1):
    return pl.pallas_call(
        _tc2_body,
        grid=(TGRID,),
        in_specs=[
            pl.BlockSpec((BLK, 1), lambda i: (i, 0)),
            pl.BlockSpec((BLK, 1), lambda i: (i, 0)),
            pl.BlockSpec((2, BLK, HID), lambda i: (0, i, 0)),
            pl.BlockSpec((BLK, HID), lambda i: (i, 0)),
            pl.BlockSpec((HID, CLS), lambda i: (0, 0)),
            pl.BlockSpec((1, HID), lambda i: (0, 0)),
        ],
        out_specs=pl.BlockSpec((BLK, CLS), lambda i: (i, 0)),
        out_shape=jax.ShapeDtypeStruct((NPAD, CLS), jnp.bfloat16),
    )(dinv, sca, part1, g1, W2, b1)


def _tc3(sca, part2, g2, b2):
    return pl.pallas_call(
        _tc3_body,
        grid=(TGRID,),
        in_specs=[
            pl.BlockSpec((BLK, 1), lambda i: (i, 0)),
            pl.BlockSpec((2, BLK, CLS), lambda i: (0, i, 0)),
            pl.BlockSpec((BLK, CLS), lambda i: (i, 0)),
            pl.BlockSpec((1, CLS), lambda i: (0, 0)),
        ],
        out_specs=pl.BlockSpec((BLK, CLS), lambda i: (i, 0)),
        out_shape=jax.ShapeDtypeStruct((N, CLS), jnp.float32),
    )(sca, part2, g2, b2)


def kernel(x, edge_index, W1, b1, W2, b2):
    pad = EPAD - E
    padv = N + (jnp.arange(pad, dtype=jnp.int32) % (NPAD - N))
    srcp = jnp.concatenate([edge_index[0], padv]).reshape(ROWS, K)
    dstp = jnp.concatenate([edge_index[1], padv]).reshape(ROWS, K)

    ones16 = jnp.ones((K, 16), jnp.float32)
    zeros16 = jnp.zeros((NPAD, 16), jnp.float32)
    zeros128 = jnp.zeros((NPAD, HID), jnp.bfloat16)
    zeros64 = jnp.zeros((NPAD, CLS), jnp.bfloat16)

    degp = _deg_kernel(dstp, ones16, zeros16)
    g1, dinv, sca = _tc1(degp, x, W1)
    part1 = _spmm128(g1, srcp, dstp, zeros128)
    g2 = _tc2(dinv, sca, part1, g1, W2, b1.reshape(1, HID))
    part2 = _spmm64(g2, srcp, dstp, zeros64)
    return _tc3(sca, part2, g2, b2.reshape(1, CLS))


# speedup vs baseline: 3.9738x; 1.0143x over previous
import functools

import jax
import jax.numpy as jnp
from jax import lax
from jax.experimental import pallas as pl
from jax.experimental.pallas import tpu as pltpu
from jax.experimental.pallas import tpu_sc as plsc

N = 10000
E = 320000
F_IN = 128
HID = 128
CLS = 64

NPAD = 10240
K = 128
NTILES = 32
STEPS = 80
EPAD = NTILES * STEPS * K
ROWS = EPAD // K
S0, S1 = 80, 80
SMAX = max(S0, S1)
RPT = NPAD // 16

_MESH = dict(core_axis_name="c", subcore_axis_name="s")

ROW_BLK = 256
GRID = NPAD // ROW_BLK


def _deg_kernel(dstp, ones16, zeros16):
    @functools.partial(
        pl.kernel,
        out_type=jax.ShapeDtypeStruct((2, NPAD, 16), jnp.float32),
        mesh=plsc.VectorSubcoreMesh(**_MESH),
        compiler_params=pltpu.CompilerParams(use_tc_tiling_on_sc=False),
        scratch_types=[
            pltpu.VMEM((STEPS, K), jnp.int32),
            pltpu.VMEM((K, 16), jnp.float32),
            pltpu.VMEM_SHARED((NPAD, 16), jnp.float32),
            pltpu.SemaphoreType.DMA,
        ],
    )
    def body(dst_hbm, ones_hbm, zeros_hbm, out_hbm, dstx, ones_v, acc, sem):
        c = lax.axis_index("c")
        s = lax.axis_index("s")
        wid = s * 2 + c
        r0 = s * RPT
        pltpu.sync_copy(zeros_hbm.at[pl.ds(r0, RPT)], acc.at[pl.ds(r0, RPT)])
        pltpu.sync_copy(ones_hbm, ones_v)
        pltpu.sync_copy(dst_hbm.at[pl.ds(wid * STEPS, STEPS)], dstx)
        plsc.subcore_barrier()

        def step(jo, carry):
            descs = [pltpu.async_copy(ones_v, acc.at[dstx.at[8 * jo + b]], sem,
                                      add=True) for b in range(8)]
            for d in descs:
                d.wait()
            return carry

        lax.fori_loop(0, STEPS // 8, step, 0)
        plsc.subcore_barrier()
        pltpu.sync_copy(acc.at[pl.ds(r0, RPT)], out_hbm.at[c, pl.ds(r0, RPT)])

    return body(dstp, ones16, zeros16)


def _make_spmm(width):
    @functools.partial(
        pl.kernel,
        out_type=jax.ShapeDtypeStruct((2, NPAD, width), jnp.bfloat16),
        mesh=plsc.VectorSubcoreMesh(**_MESH),
        compiler_params=pltpu.CompilerParams(use_tc_tiling_on_sc=False),
        scratch_types=[
            pltpu.VMEM((SMAX, K), jnp.int32),
            pltpu.VMEM((SMAX, K), jnp.int32),
            pltpu.VMEM((8, K, width), jnp.bfloat16),
            pltpu.VMEM_SHARED((NPAD, width), jnp.bfloat16),
        ] + [pltpu.SemaphoreType.DMA] * 8,
    )
    def body(g_hbm, src_hbm, dst_hbm, zeros_hbm, out_hbm, srcx, dstx, buf, acc,
             *sems):
        c = lax.axis_index("c")
        s = lax.axis_index("s")
        r0 = s * RPT
        base = jnp.where(c == 0, s * S0, 16 * S0 + s * S1)
        pltpu.sync_copy(zeros_hbm.at[pl.ds(r0, RPT)], acc.at[pl.ds(r0, RPT)])
        pltpu.sync_copy(src_hbm.at[pl.ds(base, SMAX)], srcx)
        pltpu.sync_copy(dst_hbm.at[pl.ds(base, SMAX)], dstx)
        plsc.subcore_barrier()

        NBUF = 8
        for p in range(NBUF - 1):
            pltpu.async_copy(g_hbm.at[srcx.at[p]], buf.at[p], sems[p])

        def step(jo, carry):
            for b in range(NBUF):
                j = NBUF * jo + b
                nslot = (b + NBUF - 1) % NBUF

                @pl.when(j + NBUF - 1 < S0)
                def _():
                    pltpu.async_copy(g_hbm.at[srcx.at[j + NBUF - 1]],
                                     buf.at[nslot], sems[nslot])

                pltpu.make_async_copy(g_hbm.at[srcx.at[j]], buf.at[b],
                                      sems[b]).wait()
                pltpu.sync_copy(buf.at[b], acc.at[dstx.at[j]], add=True)
            return carry

        lax.fori_loop(0, S0 // NBUF, step, 0)
        plsc.subcore_barrier()
        pltpu.sync_copy(acc.at[pl.ds(r0, RPT)], out_hbm.at[c, pl.ds(r0, RPT)])

    return body


_spmm128 = _make_spmm(HID)
_spmm64 = _make_spmm(CLS)


def _deg_dinv(degp):
    deg = 1.0 + jnp.sum(degp[0], axis=1) + jnp.sum(degp[1], axis=1)
    dinv = lax.rsqrt(deg)
    return deg, dinv


def _tc1_body(degp_ref, x_ref, w1_ref, g1_ref, dinv_ref, sca_ref):
    deg, dinv = _deg_dinv(degp_ref[...])
    g1_ref[...] = jnp.dot(dinv[:, None] * x_ref[...], w1_ref[...],
                          preferred_element_type=jnp.float32
                          ).astype(jnp.bfloat16)
    dinv_ref[...] = dinv[:, None]
    sca_ref[...] = (dinv / deg)[:, None]


def _tc2_body(dinv_ref, sca_ref, part_ref, g1_ref, w2_ref, b1_ref, g2_ref):
    p = part_ref[...].astype(jnp.float32)
    ssum = p[0] + p[1] + g1_ref[...].astype(jnp.float32)
    h1 = jnp.maximum(sca_ref[...] * ssum + b1_ref[...], 0.0)
    g2_ref[...] = (dinv_ref[...] * jnp.dot(h1, w2_ref[...],
                                           preferred_element_type=jnp.float32)
                   ).astype(jnp.bfloat16)


def _tc3_body(sca_ref, part_ref, g2_ref, b2_ref, out_ref):
    p = part_ref[...].astype(jnp.float32)
    a = (sca_ref[...] * (p[0] + p[1] + g2_ref[...].astype(jnp.float32))
         + b2_ref[...])
    m = jnp.max(a, axis=1, keepdims=True)
    ex = jnp.exp(a - m)
    out_ref[...] = (a - m) - jnp.log(jnp.sum(ex, axis=1, keepdims=True))


BLK = 400
TGRID = N // BLK


def _tc1(degp, x, W1):
    return pl.pallas_call(
        _tc1_body,
        grid=(TGRID,),
        in_specs=[
            pl.BlockSpec((2, BLK, 16), lambda i: (0, i, 0)),
            pl.BlockSpec((BLK, F_IN), lambda i: (i, 0)),
            pl.BlockSpec((F_IN, HID), lambda i: (0, 0)),
        ],
        out_specs=[
            pl.BlockSpec((BLK, HID), lambda i: (i, 0)),
            pl.BlockSpec((BLK, 1), lambda i: (i, 0)),
            pl.BlockSpec((BLK, 1), lambda i: (i, 0)),
        ],
        out_shape=[
            jax.ShapeDtypeStruct((NPAD, HID), jnp.bfloat16),
            jax.ShapeDtypeStruct((N, 1), jnp.float32),
            jax.ShapeDtypeStruct((N, 1), jnp.float32),
        ],
    )(degp, x, W1)


def _tc2(dinv, sca, part1, g1, W2, b1):
    return pl.pallas_call(
        _tc2_body,
        grid=(TGRID,),
        in_specs=[
            pl.BlockSpec((BLK, 1), lambda i: (i, 0)),
            pl.BlockSpec((BLK, 1), lambda i: (i, 0)),
            pl.BlockSpec((2, BLK, HID), lambda i: (0, i, 0)),
            pl.BlockSpec((BLK, HID), lambda i: (i, 0)),
            pl.BlockSpec((HID, CLS), lambda i: (0, 0)),
            pl.BlockSpec((1, HID), lambda i: (0, 0)),
        ],
        out_specs=pl.BlockSpec((BLK, CLS), lambda i: (i, 0)),
        out_shape=jax.ShapeDtypeStruct((NPAD, CLS), jnp.bfloat16),
    )(dinv, sca, part1, g1, W2, b1)


def _tc3(sca, part2, g2, b2):
    return pl.pallas_call(
        _tc3_body,
        grid=(TGRID,),
        in_specs=[
            pl.BlockSpec((BLK, 1), lambda i: (i, 0)),
            pl.BlockSpec((2, BLK, CLS), lambda i: (0, i, 0)),
            pl.BlockSpec((BLK, CLS), lambda i: (i, 0)),
            pl.BlockSpec((1, CLS), lambda i: (0, 0)),
        ],
        out_specs=pl.BlockSpec((BLK, CLS), lambda i: (i, 0)),
        out_shape=jax.ShapeDtypeStruct((N, CLS), jnp.float32),
    )(sca, part2, g2, b2)


def kernel(x, edge_index, W1, b1, W2, b2):
    pad = EPAD - E
    padv = N + (jnp.arange(pad, dtype=jnp.int32) % (NPAD - N))
    srcp = jnp.concatenate([edge_index[0], padv]).reshape(ROWS, K)
    dstp = jnp.concatenate([edge_index[1], padv]).reshape(ROWS, K)

    ones16 = jnp.ones((K, 16), jnp.float32)
    zeros16 = jnp.zeros((NPAD, 16), jnp.float32)
    zeros128 = jnp.zeros((NPAD, HID), jnp.bfloat16)
    zeros64 = jnp.zeros((NPAD, CLS), jnp.bfloat16)

    degp = _deg_kernel(dstp, ones16, zeros16)
    g1, dinv, sca = _tc1(degp, x, W1)
    part1 = _spmm128(g1, srcp, dstp, zeros128)
    g2 = _tc2(dinv, sca, part1, g1, W2, b1.reshape(1, HID))
    part2 = _spmm64(g2, srcp, dstp, zeros64)
    return _tc3(sca, part2, g2, b2.reshape(1, CLS))
